# baseline (speedup 1.0000x reference)
import functools
import numpy as np
import jax
import jax.numpy as jnp
from jax import lax
from jax.experimental import pallas as pl
from jax.experimental.pallas import tpu as pltpu
from jax.experimental.pallas import tpu_sc as plsc

_BINS = 30
_MOM = 0.75
_LW = 1.0
_BATCH = 16384
_NCLS = 1000

_SC_ROWS = 6144
_TC_ROWS = _BATCH - _SC_ROWS
_ROWS_BLK = 256

_NW = 32
_SC_ELEMS = _SC_ROWS * _NCLS
_PER_W = _SC_ELEMS // _NW
_CHUNK = 32000
_NCHUNK = _PER_W // _CHUNK
_NVEC = _CHUNK // 16
_UNROLL = 4

_ATH = [1.0, 1.0 / 3.0, 1.0 / 5.0, 1.0 / 7.0, 1.0 / 9.0, 1.0 / 11.0,
        1.0 / 13.0, 1.0 / 15.0]


def _tc_body(p_ref, t_ref, out_ref, cnt_ref, sum_ref, ts_ref):
    i = pl.program_id(0)
    nsteps = pl.num_programs(0)

    @pl.when(i == 0)
    def _init():
        for j in range(_BINS):
            cnt_ref[j] = 0.0
            sum_ref[j] = 0.0
        ts_ref[0] = 0.0

    x = p_ref[...]
    t = t_ref[...]
    g = jnp.abs(1.0 / (1.0 + jnp.exp(-x)) - t)
    pe = jnp.maximum(x, 0.0) - x * t + jnp.log1p(jnp.exp(-jnp.abs(x)))

    ts_ref[0] += jnp.sum(pe)
    for j in range(1, _BINS):
        e = float(np.float32(j) / np.float32(_BINS))
        mf = (g >= e).astype(jnp.float32)
        cnt_ref[j] += jnp.sum(mf)
        sum_ref[j] += jnp.sum(mf * pe)

    @pl.when(i == nsteps - 1)
    def _fin():
        tot_tc = float(_TC_ROWS * _NCLS)
        for b in range(_BINS):
            c_lo = tot_tc if b == 0 else cnt_ref[b]
            c_hi = 0.0 if b == _BINS - 1 else cnt_ref[b + 1]
            s_lo = ts_ref[0] if b == 0 else sum_ref[b]
            s_hi = 0.0 if b == _BINS - 1 else sum_ref[b + 1]
            out_ref[b] = c_lo - c_hi
            out_ref[_BINS + b] = s_lo - s_hi


def _tc_hist(preds_tc, targets_tc):
    nsteps = _TC_ROWS // _ROWS_BLK
    return pl.pallas_call(
        _tc_body,
        grid=(nsteps,),
        in_specs=[
            pl.BlockSpec((_ROWS_BLK, _NCLS), lambda i: (i, 0)),
            pl.BlockSpec((_ROWS_BLK, _NCLS), lambda i: (i, 0)),
        ],
        out_specs=pl.BlockSpec(memory_space=pltpu.SMEM),
        out_shape=jax.ShapeDtypeStruct((2 * _BINS,), jnp.float32),
        scratch_shapes=[
            pltpu.SMEM((_BINS,), jnp.float32),
            pltpu.SMEM((_BINS,), jnp.float32),
            pltpu.SMEM((1,), jnp.float32),
        ],
    )(preds_tc, targets_tc)


@functools.partial(
    pl.kernel,
    out_type=jax.ShapeDtypeStruct((_NW, 60, 16), jnp.float32),
    mesh=plsc.VectorSubcoreMesh(core_axis_name="c", subcore_axis_name="s"),
    compiler_params=pltpu.CompilerParams(needs_layout_passes=False),
    scratch_types=[
        pltpu.VMEM((_CHUNK,), jnp.float32),
        pltpu.VMEM((_CHUNK,), jnp.float32),
        pltpu.VMEM((60, 16), jnp.float32),
        pltpu.VMEM((60, 16), jnp.float32),
        pltpu.VMEM((60, 16), jnp.float32),
        pltpu.VMEM((60, 16), jnp.float32),
    ],
)
def _sc_hist(preds_hbm, targets_hbm, out_hbm, xbuf, tbuf, h0, h1, h2, h3):
    banks = [h0, h1, h2, h3]
    wid = lax.axis_index("c") * 16 + lax.axis_index("s")
    base = wid * _PER_W

    i16 = lax.iota(jnp.int32, 16)
    lane = i16
    ones = jnp.full((16,), 1.0, dtype=jnp.float32)
    zeros = jnp.zeros((16,), dtype=jnp.float32)

    for r in range(60):
        for hb in banks:
            hb[r, :] = zeros

    def chunk_body(c, _):
        off = base + c * _CHUNK
        pltpu.sync_copy(preds_hbm.at[pl.ds(off, _CHUNK)], xbuf)
        pltpu.sync_copy(targets_hbm.at[pl.ds(off, _CHUNK)], tbuf)

        @plsc.parallel_loop(0, _NVEC // _UNROLL, 1, unroll=2)
        def vec_body(v):
            base_o = v * (16 * _UNROLL)
            for uu in range(_UNROLL):
                o = base_o + uu * 16
                x = xbuf[pl.ds(o, 16)]
                t = tbuf[pl.ds(o, 16)]
                ax = jnp.abs(x)
                u = jnp.exp(-ax)
                s = jnp.where(x >= 0.0, ones, u) / (1.0 + u)
                g = jnp.abs(s - t)
                k0 = jnp.minimum((g * 30.0).astype(jnp.int32), 29)
                k0f = k0.astype(jnp.float32)
                e_lo = k0f / 30.0
                e_hi = jnp.where(k0 >= 29, jnp.inf, (k0f + 1.0) / 30.0)
                k = (k0 + jnp.where(g >= e_hi, 1, 0)
                     - jnp.where(g < e_lo, 1, 0))
                w = u / (2.0 + u)
                w2 = w * w
                p = jnp.full((16,), _ATH[-1], dtype=jnp.float32)
                for coef in _ATH[-2::-1]:
                    p = p * w2 + coef
                l1p = (2.0 * w) * p
                pe = jnp.maximum(x, 0.0) - x * t + l1p
                hb = banks[uu]
                plsc.addupdate_scatter(hb, [k, lane], ones)
                plsc.addupdate_scatter(hb, [k + _BINS, lane], pe)

        return _

    lax.fori_loop(0, _NCHUNK, chunk_body, None)
    for r in range(60):
        h0[r, :] = ((h0[r, :] + h1[r, :]) + (h2[r, :] + h3[r, :]))
    pltpu.sync_copy(h0, out_hbm.at[wid])


def kernel(preds, targets, acc_sum):
    h = _sc_hist(preds[:_SC_ROWS].reshape(-1), targets[:_SC_ROWS].reshape(-1))
    tc = _tc_hist(preds[_SC_ROWS:], targets[_SC_ROWS:])
    counts = h[:, :_BINS, :].sum(axis=(0, 2)) + tc[:_BINS]
    sums = h[:, _BINS:, :].sum(axis=(0, 2)) + tc[_BINS:]
    tot = float(_BATCH * _NCLS)
    ne = counts > 0.0
    acc_new = jnp.where(ne, _MOM * acc_sum + (1.0 - _MOM) * counts, acc_sum)
    bin_w = jnp.where(ne, tot / jnp.where(ne, acc_new, 1.0), 0.0)
    n = jnp.sum(ne.astype(jnp.float32))
    loss = jnp.sum(bin_w * sums) / tot
    loss = jnp.where(n > 0.0, loss / jnp.maximum(n, 1.0), loss)
    return loss * _LW


# speedup vs baseline: 1.1452x; 1.1452x over previous
import functools
import numpy as np
import jax
import jax.numpy as jnp
from jax import lax
from jax.experimental import pallas as pl
from jax.experimental.pallas import tpu as pltpu
from jax.experimental.pallas import tpu_sc as plsc

_BINS = 30
_MOM = 0.75
_LW = 1.0
_BATCH = 16384
_NCLS = 1000

_SC_ROWS = 10240
_TC_ROWS = _BATCH - _SC_ROWS
_ROWS_BLK = 256

_NW = 32
_SC_ELEMS = _SC_ROWS * _NCLS
_PER_W = _SC_ELEMS // _NW
_CHUNK = 32000
_NCHUNK = _PER_W // _CHUNK
_NVEC = _CHUNK // 16
_UNROLL = 4

_ATH = [1.0, 1.0 / 3.0, 1.0 / 5.0, 1.0 / 7.0, 1.0 / 9.0, 1.0 / 11.0,
        1.0 / 13.0, 1.0 / 15.0]


def _tc_body(p_ref, t_ref, out_ref, cnt_ref, sum_ref, ts_ref):
    i = pl.program_id(0)
    nsteps = pl.num_programs(0)

    @pl.when(i == 0)
    def _init():
        for j in range(_BINS):
            cnt_ref[j] = 0.0
            sum_ref[j] = 0.0
        ts_ref[0] = 0.0

    x = p_ref[...]
    t = t_ref[...]
    g = jnp.abs(1.0 / (1.0 + jnp.exp(-x)) - t)
    pe = jnp.maximum(x, 0.0) - x * t + jnp.log1p(jnp.exp(-jnp.abs(x)))

    ts_ref[0] += jnp.sum(pe)
    for j in range(1, _BINS):
        e = float(np.float32(j) / np.float32(_BINS))
        mf = (g >= e).astype(jnp.float32)
        cnt_ref[j] += jnp.sum(mf)
        sum_ref[j] += jnp.sum(mf * pe)

    @pl.when(i == nsteps - 1)
    def _fin():
        tot_tc = float(_TC_ROWS * _NCLS)
        for b in range(_BINS):
            c_lo = tot_tc if b == 0 else cnt_ref[b]
            c_hi = 0.0 if b == _BINS - 1 else cnt_ref[b + 1]
            s_lo = ts_ref[0] if b == 0 else sum_ref[b]
            s_hi = 0.0 if b == _BINS - 1 else sum_ref[b + 1]
            out_ref[b] = c_lo - c_hi
            out_ref[_BINS + b] = s_lo - s_hi


def _tc_hist(preds_tc, targets_tc):
    nsteps = _TC_ROWS // _ROWS_BLK
    return pl.pallas_call(
        _tc_body,
        grid=(nsteps,),
        in_specs=[
            pl.BlockSpec((_ROWS_BLK, _NCLS), lambda i: (i, 0)),
            pl.BlockSpec((_ROWS_BLK, _NCLS), lambda i: (i, 0)),
        ],
        out_specs=pl.BlockSpec(memory_space=pltpu.SMEM),
        out_shape=jax.ShapeDtypeStruct((2 * _BINS,), jnp.float32),
        scratch_shapes=[
            pltpu.SMEM((_BINS,), jnp.float32),
            pltpu.SMEM((_BINS,), jnp.float32),
            pltpu.SMEM((1,), jnp.float32),
        ],
    )(preds_tc, targets_tc)


@functools.partial(
    pl.kernel,
    out_type=jax.ShapeDtypeStruct((_NW, 60, 16), jnp.float32),
    mesh=plsc.VectorSubcoreMesh(core_axis_name="c", subcore_axis_name="s"),
    compiler_params=pltpu.CompilerParams(needs_layout_passes=False),
    scratch_types=[
        pltpu.VMEM((_CHUNK,), jnp.float32),
        pltpu.VMEM((_CHUNK,), jnp.float32),
        pltpu.VMEM((60, 16), jnp.float32),
        pltpu.VMEM((60, 16), jnp.float32),
        pltpu.VMEM((60, 16), jnp.float32),
        pltpu.VMEM((60, 16), jnp.float32),
    ],
)
def _sc_hist(preds_hbm, targets_hbm, out_hbm, xbuf, tbuf, h0, h1, h2, h3):
    banks = [h0, h1, h2, h3]
    wid = lax.axis_index("c") * 16 + lax.axis_index("s")
    base = wid * _PER_W

    i16 = lax.iota(jnp.int32, 16)
    lane = i16
    ones = jnp.full((16,), 1.0, dtype=jnp.float32)
    zeros = jnp.zeros((16,), dtype=jnp.float32)

    for r in range(60):
        for hb in banks:
            hb[r, :] = zeros

    def chunk_body(c, _):
        off = base + c * _CHUNK
        pltpu.sync_copy(preds_hbm.at[pl.ds(off, _CHUNK)], xbuf)
        pltpu.sync_copy(targets_hbm.at[pl.ds(off, _CHUNK)], tbuf)

        @plsc.parallel_loop(0, _NVEC // _UNROLL, 1, unroll=2)
        def vec_body(v):
            base_o = v * (16 * _UNROLL)
            for uu in range(_UNROLL):
                o = base_o + uu * 16
                x = xbuf[pl.ds(o, 16)]
                t = tbuf[pl.ds(o, 16)]
                ax = jnp.abs(x)
                u = jnp.exp(-ax)
                s = jnp.where(x >= 0.0, ones, u) / (1.0 + u)
                g = jnp.abs(s - t)
                k0 = jnp.minimum((g * 30.0).astype(jnp.int32), 29)
                k0f = k0.astype(jnp.float32)
                e_lo = k0f / 30.0
                e_hi = jnp.where(k0 >= 29, jnp.inf, (k0f + 1.0) / 30.0)
                k = (k0 + jnp.where(g >= e_hi, 1, 0)
                     - jnp.where(g < e_lo, 1, 0))
                w = u / (2.0 + u)
                w2 = w * w
                p = jnp.full((16,), _ATH[-1], dtype=jnp.float32)
                for coef in _ATH[-2::-1]:
                    p = p * w2 + coef
                l1p = (2.0 * w) * p
                pe = jnp.maximum(x, 0.0) - x * t + l1p
                hb = banks[uu]
                plsc.addupdate_scatter(hb, [k, lane], ones)
                plsc.addupdate_scatter(hb, [k + _BINS, lane], pe)

        return _

    lax.fori_loop(0, _NCHUNK, chunk_body, None)
    for r in range(60):
        h0[r, :] = ((h0[r, :] + h1[r, :]) + (h2[r, :] + h3[r, :]))
    pltpu.sync_copy(h0, out_hbm.at[wid])


def kernel(preds, targets, acc_sum):
    h = _sc_hist(preds[:_SC_ROWS].reshape(-1), targets[:_SC_ROWS].reshape(-1))
    tc = _tc_hist(preds[_SC_ROWS:], targets[_SC_ROWS:])
    counts = h[:, :_BINS, :].sum(axis=(0, 2)) + tc[:_BINS]
    sums = h[:, _BINS:, :].sum(axis=(0, 2)) + tc[_BINS:]
    tot = float(_BATCH * _NCLS)
    ne = counts > 0.0
    acc_new = jnp.where(ne, _MOM * acc_sum + (1.0 - _MOM) * counts, acc_sum)
    bin_w = jnp.where(ne, tot / jnp.where(ne, acc_new, 1.0), 0.0)
    n = jnp.sum(ne.astype(jnp.float32))
    loss = jnp.sum(bin_w * sums) / tot
    loss = jnp.where(n > 0.0, loss / jnp.maximum(n, 1.0), loss)
    return loss * _LW
